# Initial kernel scaffold; baseline (speedup 1.0000x reference)
import jax
import jax.numpy as jnp
from jax.experimental import pallas as pl


def kernel(spatial_nodes, sphere_nodes, edges, senders, receivers, edge_W1a, edge_W1b, edge_b1, edge_ln_s, edge_ln_o, edge_W2, edge_b2, node_W1, node_b1, node_ln_s, node_ln_o, node_W2, node_b2):
    raise NotImplementedError("write your pallas kernel here")


# speedup vs baseline: 1.9140x; 1.9140x over previous
import functools

import jax
import jax.numpy as jnp
from jax import lax
from jax.experimental import pallas as pl
from jax.experimental.pallas import tpu as pltpu
from jax.experimental.pallas import tpu_sc as plsc

N_SPATIAL = 65160
N_SPHERE = 10242
E = 61452
D = 128
STEPS_ = 3

NC = 2
NS = 16
NW = NC * NS

CHUNK = 128
CHUNKS_PER_W = 16
PER_W = CHUNK * CHUNKS_PER_W
E_PAD = PER_W * NW

N_ACC = 10368
ACC_PER_TILE = N_ACC // NS
DUMMY = N_SPHERE

BE = 4096
BN = 1152


@functools.lru_cache(maxsize=None)
def _sc_gather_op():
    mesh = plsc.VectorSubcoreMesh(core_axis_name="c", subcore_axis_name="s",
                                  num_cores=NC, num_subcores=NS)

    @functools.partial(
        pl.kernel,
        out_type=jax.ShapeDtypeStruct((E_PAD, D), jnp.float32),
        mesh=mesh,
        scratch_types=[
            pltpu.VMEM((PER_W,), jnp.int32),
            pltpu.VMEM((CHUNK, D), jnp.float32),
            pltpu.SemaphoreType.DMA,
        ],
    )
    def gather_kernel(table, idx, out, idx_v, rows_v, sem):
        wid = lax.axis_index("s") * NC + lax.axis_index("c")
        base = wid * PER_W
        pltpu.sync_copy(idx.at[pl.ds(base, PER_W)], idx_v)

        def body(j, carry):
            off = j * CHUNK
            pltpu.async_copy(table.at[idx_v.at[pl.ds(off, CHUNK)]],
                             rows_v, sem).wait()
            pltpu.sync_copy(rows_v, out.at[pl.ds(base + off, CHUNK)])
            return carry

        lax.fori_loop(0, CHUNKS_PER_W, body, 0)

    return gather_kernel


@functools.lru_cache(maxsize=None)
def _sc_scatter_op():
    mesh = plsc.VectorSubcoreMesh(core_axis_name="c", subcore_axis_name="s",
                                  num_cores=NC, num_subcores=NS)

    @functools.partial(
        pl.kernel,
        out_type=jax.ShapeDtypeStruct((NC * N_ACC, D), jnp.float32),
        mesh=mesh,
        scratch_types=[
            pltpu.VMEM((CHUNK,), jnp.int32),
            pltpu.VMEM((CHUNK, D), jnp.float32),
            pltpu.VMEM_SHARED((N_ACC, D), jnp.float32),
            pltpu.SemaphoreType.DMA,
        ],
    )
    def scatter_kernel(vals, idx, zeros, out, idx_v, rows_v, acc, sem):
        cid = lax.axis_index("c")
        sid = lax.axis_index("s")
        row0 = sid * ACC_PER_TILE
        pltpu.sync_copy(zeros.at[pl.ds(row0, ACC_PER_TILE)],
                        acc.at[pl.ds(row0, ACC_PER_TILE)])
        plsc.subcore_barrier()

        base = (cid * NS + sid) * PER_W

        def body(j, carry):
            off = base + j * CHUNK
            pltpu.sync_copy(idx.at[pl.ds(off, CHUNK)], idx_v)
            pltpu.sync_copy(vals.at[pl.ds(off, CHUNK)], rows_v)
            pltpu.sync_copy(rows_v, acc.at[idx_v], add=True)
            return carry

        lax.fori_loop(0, CHUNKS_PER_W, body, 0)
        plsc.subcore_barrier()
        pltpu.sync_copy(acc.at[pl.ds(row0, ACC_PER_TILE)],
                        out.at[pl.ds(cid * N_ACC + row0, ACC_PER_TILE)])

    return scatter_kernel


def _gather(table, idx):
    return _sc_gather_op()(table, idx)


def _scatter(vals, idx, zeros):
    return _sc_scatter_op()(vals, idx, zeros)


def _row_spec(r):
    return pl.BlockSpec((r, D), lambda i: (i, 0))


def _w_spec(r):
    return pl.BlockSpec((r, D), lambda i: (0, 0))


def _ln_relu(pre, lns, lno):
    x = jnp.maximum(pre, 0.0)
    mu = jnp.mean(x, axis=1, keepdims=True)
    xm = x - mu
    var = jnp.mean(xm * xm, axis=1, keepdims=True)
    return xm * lax.rsqrt(var + 1e-5) * lns + lno


def _dot(a, b):
    return jnp.dot(a, b, preferred_element_type=jnp.float32)


def _pre_body(sf_ref, ed_ref, was_ref, wae_ref, wbs_ref, b1_ref, ba_ref, bb_ref):
    sf = sf_ref[...]
    b1 = b1_ref[...]
    ba_ref[...] = _dot(ed_ref[...], wae_ref[...]) + _dot(sf, was_ref[...]) + b1
    bb_ref[...] = _dot(sf, wbs_ref[...]) + b1


def _precompute(s_feat, edges8, w1as, w1ae8, w1bs, b1r):
    return pl.pallas_call(
        _pre_body,
        grid=(E_PAD // BE,),
        in_specs=[_row_spec(BE), pl.BlockSpec((BE, 8), lambda i: (i, 0)),
                  _w_spec(D), pl.BlockSpec((8, D), lambda i: (0, 0)),
                  _w_spec(D), _w_spec(1)],
        out_specs=[_row_spec(BE), _row_spec(BE)],
        out_shape=[jax.ShapeDtypeStruct((E_PAD, D), jnp.float32)] * 2,
    )(s_feat, edges8, w1as, w1ae8, w1bs, b1r)


def _proj_body(x_ref, w_ref, o_ref):
    o_ref[...] = _dot(x_ref[...], w_ref[...])


def _proj(x, w):
    r = x.shape[0]
    return pl.pallas_call(
        _proj_body,
        grid=(r // BN,),
        in_specs=[_row_spec(BN), _w_spec(D)],
        out_specs=_row_spec(BN),
        out_shape=jax.ShapeDtypeStruct((r, D), jnp.float32),
    )(x, w)


def _edge_full_body(base_ref, bb_ref, g_ref, w2_ref, w1e_ref, b2_ref,
                    lns_ref, lno_ref, e_ref, bn_ref):
    h = _ln_relu(base_ref[...] + g_ref[...], lns_ref[...], lno_ref[...])
    e = _dot(h, w2_ref[...]) + b2_ref[...]
    e_ref[...] = e
    bn_ref[...] = bb_ref[...] + _dot(e, w1e_ref[...])


def _edge_full(base, bb, g, w2, w1e, b2r, lnsr, lnor):
    return pl.pallas_call(
        _edge_full_body,
        grid=(E_PAD // BE,),
        in_specs=[_row_spec(BE), _row_spec(BE), _row_spec(BE),
                  _w_spec(D), _w_spec(D), _w_spec(1), _w_spec(1), _w_spec(1)],
        out_specs=[_row_spec(BE), _row_spec(BE)],
        out_shape=[jax.ShapeDtypeStruct((E_PAD, D), jnp.float32)] * 2,
    )(base, bb, g, w2, w1e, b2r, lnsr, lnor)


def _edge_last_body(base_ref, g_ref, w2_ref, b2_ref, lns_ref, lno_ref, e_ref):
    h = _ln_relu(base_ref[...] + g_ref[...], lns_ref[...], lno_ref[...])
    e_ref[...] = _dot(h, w2_ref[...]) + b2_ref[...]


def _edge_last(base, g, w2, b2r, lnsr, lnor):
    return pl.pallas_call(
        _edge_last_body,
        grid=(E_PAD // BE,),
        in_specs=[_row_spec(BE), _row_spec(BE),
                  _w_spec(D), _w_spec(1), _w_spec(1), _w_spec(1)],
        out_specs=_row_spec(BE),
        out_shape=jax.ShapeDtypeStruct((E_PAD, D), jnp.float32),
    )(base, g, w2, b2r, lnsr, lnor)


def _node_body(sph_ref, m0_ref, m1_ref, w1t_ref, w1b_ref, b1_ref,
               lns_ref, lno_ref, w2_ref, b2_ref, wr_ref, out_ref, p_ref):
    msg = m0_ref[...] + m1_ref[...]
    pre = _dot(sph_ref[...], w1t_ref[...]) + _dot(msg, w1b_ref[...]) + b1_ref[...]
    h = _ln_relu(pre, lns_ref[...], lno_ref[...])
    upd = _dot(h, w2_ref[...]) + b2_ref[...]
    out_ref[...] = upd
    p_ref[...] = _dot(upd, wr_ref[...])


def _node(sphere, m0, m1, w1t, w1b, b1r, lnsr, lnor, w2, b2r, wr):
    return pl.pallas_call(
        _node_body,
        grid=(N_ACC // BN,),
        in_specs=[_row_spec(BN), _row_spec(BN), _row_spec(BN),
                  _w_spec(D), _w_spec(D), _w_spec(1), _w_spec(1), _w_spec(1),
                  _w_spec(D), _w_spec(1), _w_spec(D)],
        out_specs=[_row_spec(BN), _row_spec(BN)],
        out_shape=[jax.ShapeDtypeStruct((N_ACC, D), jnp.float32)] * 2,
    )(sphere, m0, m1, w1t, w1b, b1r, lnsr, lnor, w2, b2r, wr)


def kernel(spatial_nodes, sphere_nodes, edges, senders, receivers,
           edge_W1a, edge_W1b, edge_b1, edge_ln_s, edge_ln_o, edge_W2, edge_b2,
           node_W1, node_b1, node_ln_s, node_ln_o, node_W2, node_b2):
    f32 = jnp.float32
    w1a_e, w1a_s, w1a_r = edge_W1a[:3], edge_W1a[3:3 + D], edge_W1a[3 + D:]
    w1b_e, w1b_s, w1b_r = edge_W1b[:D], edge_W1b[D:2 * D], edge_W1b[2 * D:]
    w1ae8 = jnp.zeros((8, D), f32).at[:3].set(w1a_e)
    nw1_t, nw1_b = node_W1[:D], node_W1[D:]

    edges8 = jnp.zeros((E_PAD, 8), f32).at[:E, :3].set(edges)
    send_pad = jnp.zeros((E_PAD,), jnp.int32).at[:E].set(senders)
    recv_pad = jnp.full((E_PAD,), DUMMY, jnp.int32).at[:E].set(receivers)
    sphere_pad = jnp.zeros((N_ACC, D), f32).at[:N_SPHERE].set(sphere_nodes)
    zeros_acc = jnp.zeros((N_ACC, D), f32)

    b1r = edge_b1.reshape(1, D)
    b2r = edge_b2.reshape(1, D)
    lnsr = edge_ln_s.reshape(1, D)
    lnor = edge_ln_o.reshape(1, D)
    nb1r = node_b1.reshape(1, D)
    nb2r = node_b2.reshape(1, D)
    nlnsr = node_ln_s.reshape(1, D)
    nlnor = node_ln_o.reshape(1, D)

    s_feat = _gather(spatial_nodes, send_pad)
    base, bb = _precompute(s_feat, edges8, w1a_s, w1ae8, w1b_s, b1r)
    proj = _proj(sphere_pad, w1a_r)
    sphere = sphere_pad
    for t in range(STEPS_):
        g = _gather(proj, recv_pad)
        if t < STEPS_ - 1:
            e_out, base = _edge_full(base, bb, g, edge_W2, w1b_e,
                                     b2r, lnsr, lnor)
        else:
            e_out = _edge_last(base, g, edge_W2, b2r, lnsr, lnor)
        acc = _scatter(e_out, recv_pad, zeros_acc)
        sphere, proj = _node(sphere, acc[:N_ACC], acc[N_ACC:],
                             nw1_t, nw1_b, nb1r, nlnsr, nlnor,
                             node_W2, nb2r, w1b_r)
    return sphere[:N_SPHERE]


# speedup vs baseline: 3.8991x; 2.0371x over previous
import functools

import jax
import jax.numpy as jnp
from jax import lax
from jax.experimental import pallas as pl
from jax.experimental.pallas import tpu as pltpu
from jax.experimental.pallas import tpu_sc as plsc

N_SPATIAL = 65160
N_SPHERE = 10242
E = 61452
D = 128
STEPS_ = 3

NC = 2
NS = 16
NW = NC * NS

CHUNK = 128
CHUNKS_PER_W = 16
PER_W = CHUNK * CHUNKS_PER_W
E_PAD = PER_W * NW

N_ACC = 10368
ACC_PER_TILE = N_ACC // NS
DUMMY = N_SPHERE

BE = 4096
BN = 1152


NBUF = 4
NBUF_S = 2


@functools.lru_cache(maxsize=None)
def _sc_gather_op():
    mesh = plsc.VectorSubcoreMesh(core_axis_name="c", subcore_axis_name="s",
                                  num_cores=NC, num_subcores=NS)

    @functools.partial(
        pl.kernel,
        out_type=jax.ShapeDtypeStruct((E_PAD, D), jnp.float32),
        mesh=mesh,
        scratch_types=[
            pltpu.VMEM((PER_W,), jnp.int32),
            pltpu.VMEM((NBUF, CHUNK, D), jnp.float32),
        ] + [pltpu.SemaphoreType.DMA] * NBUF,
    )
    def gather_kernel(table, idx, out, idx_v, bufs, *sems):
        wid = lax.axis_index("s") * NC + lax.axis_index("c")
        base = wid * PER_W
        pltpu.sync_copy(idx.at[pl.ds(base, PER_W)], idx_v)

        def start(j, k):
            pltpu.async_copy(table.at[idx_v.at[pl.ds(j * CHUNK, CHUNK)]],
                             bufs.at[k], sems[k])

        def drain(k):
            pltpu.make_async_copy(table.at[idx_v.at[pl.ds(0, CHUNK)]],
                                  bufs.at[k], sems[k]).wait()

        for k in range(NBUF):
            start(k, k)

        def body(i, carry):
            for k in range(NBUF):
                j = i * NBUF + k
                drain(k)
                pltpu.sync_copy(bufs.at[k],
                                out.at[pl.ds(base + j * CHUNK, CHUNK)])
                start(j + NBUF, k)
            return carry

        lax.fori_loop(0, CHUNKS_PER_W // NBUF - 1, body, 0)
        for k in range(NBUF):
            j = CHUNKS_PER_W - NBUF + k
            drain(k)
            pltpu.sync_copy(bufs.at[k], out.at[pl.ds(base + j * CHUNK, CHUNK)])

    return gather_kernel


@functools.lru_cache(maxsize=None)
def _sc_scatter_op():
    mesh = plsc.VectorSubcoreMesh(core_axis_name="c", subcore_axis_name="s",
                                  num_cores=NC, num_subcores=NS)

    @functools.partial(
        pl.kernel,
        out_type=jax.ShapeDtypeStruct((NC * N_ACC, D), jnp.float32),
        mesh=mesh,
        scratch_types=[
            pltpu.VMEM((CHUNKS_PER_W, CHUNK), jnp.int32),
            pltpu.VMEM((NBUF_S, CHUNK, D), jnp.float32),
            pltpu.VMEM_SHARED((N_ACC, D), jnp.float32),
        ] + [pltpu.SemaphoreType.DMA] * NBUF_S,
    )
    def scatter_kernel(vals, idx3, zeros, out, idx_v, bufs, acc, *sems):
        cid = lax.axis_index("c")
        sid = lax.axis_index("s")
        row0 = sid * ACC_PER_TILE
        pltpu.sync_copy(zeros.at[pl.ds(row0, ACC_PER_TILE)],
                        acc.at[pl.ds(row0, ACC_PER_TILE)])
        wid = cid * NS + sid
        base = wid * PER_W
        pltpu.sync_copy(idx3.at[wid], idx_v)
        plsc.subcore_barrier()

        def start(j, k):
            pltpu.async_copy(vals.at[pl.ds(base + j * CHUNK, CHUNK)],
                             bufs.at[k], sems[k])

        def drain(k):
            pltpu.make_async_copy(vals.at[pl.ds(base, CHUNK)],
                                  bufs.at[k], sems[k]).wait()

        for k in range(NBUF_S):
            start(k, k)

        def body(i, carry):
            for k in range(NBUF_S):
                j = i * NBUF_S + k
                drain(k)
                pltpu.sync_copy(bufs.at[k], acc.at[idx_v.at[j]], add=True)
                start(j + NBUF_S, k)
            return carry

        lax.fori_loop(0, CHUNKS_PER_W // NBUF_S - 1, body, 0)
        for k in range(NBUF_S):
            j = CHUNKS_PER_W - NBUF_S + k
            drain(k)
            pltpu.sync_copy(bufs.at[k], acc.at[idx_v.at[j]], add=True)

        plsc.subcore_barrier()
        pltpu.sync_copy(acc.at[pl.ds(row0, ACC_PER_TILE)],
                        out.at[pl.ds(cid * N_ACC + row0, ACC_PER_TILE)])

    return scatter_kernel


def _gather(table, idx):
    return _sc_gather_op()(table, idx)


def _scatter(vals, idx3, zeros):
    return _sc_scatter_op()(vals, idx3, zeros)


def _row_spec(r):
    return pl.BlockSpec((r, D), lambda i: (i, 0))


def _w_spec(r):
    return pl.BlockSpec((r, D), lambda i: (0, 0))


def _ln_relu(pre, lns, lno):
    x = jnp.maximum(pre, 0.0)
    mu = jnp.mean(x, axis=1, keepdims=True)
    xm = x - mu
    var = jnp.mean(xm * xm, axis=1, keepdims=True)
    return xm * lax.rsqrt(var + 1e-5) * lns + lno


def _dot(a, b):
    return jnp.dot(a, b, preferred_element_type=jnp.float32)


def _pre_body(sf_ref, ed_ref, was_ref, wae_ref, wbs_ref, b1_ref, ba_ref, bb_ref):
    sf = sf_ref[...]
    b1 = b1_ref[...]
    ba_ref[...] = _dot(ed_ref[...], wae_ref[...]) + _dot(sf, was_ref[...]) + b1
    bb_ref[...] = _dot(sf, wbs_ref[...]) + b1


def _precompute(s_feat, edges8, w1as, w1ae8, w1bs, b1r):
    return pl.pallas_call(
        _pre_body,
        grid=(E_PAD // BE,),
        in_specs=[_row_spec(BE), pl.BlockSpec((BE, 8), lambda i: (i, 0)),
                  _w_spec(D), pl.BlockSpec((8, D), lambda i: (0, 0)),
                  _w_spec(D), _w_spec(1)],
        out_specs=[_row_spec(BE), _row_spec(BE)],
        out_shape=[jax.ShapeDtypeStruct((E_PAD, D), jnp.float32)] * 2,
    )(s_feat, edges8, w1as, w1ae8, w1bs, b1r)


def _proj_body(x_ref, w_ref, o_ref):
    o_ref[...] = _dot(x_ref[...], w_ref[...])


def _proj(x, w):
    r = x.shape[0]
    return pl.pallas_call(
        _proj_body,
        grid=(r // BN,),
        in_specs=[_row_spec(BN), _w_spec(D)],
        out_specs=_row_spec(BN),
        out_shape=jax.ShapeDtypeStruct((r, D), jnp.float32),
    )(x, w)


def _edge_full_body(base_ref, bb_ref, g_ref, w2_ref, w1e_ref, b2_ref,
                    lns_ref, lno_ref, e_ref, bn_ref):
    h = _ln_relu(base_ref[...] + g_ref[...], lns_ref[...], lno_ref[...])
    e = _dot(h, w2_ref[...]) + b2_ref[...]
    e_ref[...] = e
    bn_ref[...] = bb_ref[...] + _dot(e, w1e_ref[...])


def _edge_full(base, bb, g, w2, w1e, b2r, lnsr, lnor):
    return pl.pallas_call(
        _edge_full_body,
        grid=(E_PAD // BE,),
        in_specs=[_row_spec(BE), _row_spec(BE), _row_spec(BE),
                  _w_spec(D), _w_spec(D), _w_spec(1), _w_spec(1), _w_spec(1)],
        out_specs=[_row_spec(BE), _row_spec(BE)],
        out_shape=[jax.ShapeDtypeStruct((E_PAD, D), jnp.float32)] * 2,
    )(base, bb, g, w2, w1e, b2r, lnsr, lnor)


def _edge_last_body(base_ref, g_ref, w2_ref, b2_ref, lns_ref, lno_ref, e_ref):
    h = _ln_relu(base_ref[...] + g_ref[...], lns_ref[...], lno_ref[...])
    e_ref[...] = _dot(h, w2_ref[...]) + b2_ref[...]


def _edge_last(base, g, w2, b2r, lnsr, lnor):
    return pl.pallas_call(
        _edge_last_body,
        grid=(E_PAD // BE,),
        in_specs=[_row_spec(BE), _row_spec(BE),
                  _w_spec(D), _w_spec(1), _w_spec(1), _w_spec(1)],
        out_specs=_row_spec(BE),
        out_shape=jax.ShapeDtypeStruct((E_PAD, D), jnp.float32),
    )(base, g, w2, b2r, lnsr, lnor)


def _node_body(sph_ref, m0_ref, m1_ref, w1t_ref, w1b_ref, b1_ref,
               lns_ref, lno_ref, w2_ref, b2_ref, wr_ref, out_ref, p_ref):
    msg = m0_ref[...] + m1_ref[...]
    pre = _dot(sph_ref[...], w1t_ref[...]) + _dot(msg, w1b_ref[...]) + b1_ref[...]
    h = _ln_relu(pre, lns_ref[...], lno_ref[...])
    upd = _dot(h, w2_ref[...]) + b2_ref[...]
    out_ref[...] = upd
    p_ref[...] = _dot(upd, wr_ref[...])


def _node(sphere, m0, m1, w1t, w1b, b1r, lnsr, lnor, w2, b2r, wr):
    return pl.pallas_call(
        _node_body,
        grid=(N_ACC // BN,),
        in_specs=[_row_spec(BN), _row_spec(BN), _row_spec(BN),
                  _w_spec(D), _w_spec(D), _w_spec(1), _w_spec(1), _w_spec(1),
                  _w_spec(D), _w_spec(1), _w_spec(D)],
        out_specs=[_row_spec(BN), _row_spec(BN)],
        out_shape=[jax.ShapeDtypeStruct((N_ACC, D), jnp.float32)] * 2,
    )(sphere, m0, m1, w1t, w1b, b1r, lnsr, lnor, w2, b2r, wr)


def kernel(spatial_nodes, sphere_nodes, edges, senders, receivers,
           edge_W1a, edge_W1b, edge_b1, edge_ln_s, edge_ln_o, edge_W2, edge_b2,
           node_W1, node_b1, node_ln_s, node_ln_o, node_W2, node_b2):
    f32 = jnp.float32
    w1a_e, w1a_s, w1a_r = edge_W1a[:3], edge_W1a[3:3 + D], edge_W1a[3 + D:]
    w1b_e, w1b_s, w1b_r = edge_W1b[:D], edge_W1b[D:2 * D], edge_W1b[2 * D:]
    w1ae8 = jnp.zeros((8, D), f32).at[:3].set(w1a_e)
    nw1_t, nw1_b = node_W1[:D], node_W1[D:]

    edges8 = jnp.zeros((E_PAD, 8), f32).at[:E, :3].set(edges)
    send_pad = jnp.zeros((E_PAD,), jnp.int32).at[:E].set(senders)
    recv_pad = jnp.full((E_PAD,), DUMMY, jnp.int32).at[:E].set(receivers)
    recv3 = recv_pad.reshape(NW, CHUNKS_PER_W, CHUNK)
    sphere_pad = jnp.zeros((N_ACC, D), f32).at[:N_SPHERE].set(sphere_nodes)
    zeros_acc = jnp.zeros((N_ACC, D), f32)

    b1r = edge_b1.reshape(1, D)
    b2r = edge_b2.reshape(1, D)
    lnsr = edge_ln_s.reshape(1, D)
    lnor = edge_ln_o.reshape(1, D)
    nb1r = node_b1.reshape(1, D)
    nb2r = node_b2.reshape(1, D)
    nlnsr = node_ln_s.reshape(1, D)
    nlnor = node_ln_o.reshape(1, D)

    s_feat = _gather(spatial_nodes, send_pad)
    base, bb = _precompute(s_feat, edges8, w1a_s, w1ae8, w1b_s, b1r)
    proj = _proj(sphere_pad, w1a_r)
    sphere = sphere_pad
    for t in range(STEPS_):
        g = _gather(proj, recv_pad)
        if t < STEPS_ - 1:
            e_out, base = _edge_full(base, bb, g, edge_W2, w1b_e,
                                     b2r, lnsr, lnor)
        else:
            e_out = _edge_last(base, g, edge_W2, b2r, lnsr, lnor)
        acc = _scatter(e_out, recv3, zeros_acc)
        sphere, proj = _node(sphere, acc[:N_ACC], acc[N_ACC:],
                             nw1_t, nw1_b, nb1r, nlnsr, nlnor,
                             node_W2, nb2r, w1b_r)
    return sphere[:N_SPHERE]


# speedup vs baseline: 3.9013x; 1.0006x over previous
import functools

import jax
import jax.numpy as jnp
from jax import lax
from jax.experimental import pallas as pl
from jax.experimental.pallas import tpu as pltpu
from jax.experimental.pallas import tpu_sc as plsc

N_SPATIAL = 65160
N_SPHERE = 10242
E = 61452
D = 128
STEPS_ = 3

NC = 2
NS = 16
NW = NC * NS

CHUNK = 128
CHUNKS_PER_W = 16
PER_W = CHUNK * CHUNKS_PER_W
E_PAD = PER_W * NW

N_ACC = 10368
ACC_PER_TILE = N_ACC // NS
DUMMY = N_SPHERE

BE = 4096
BN = 1152


NBUF = 4
NBUF_S = 2


@functools.lru_cache(maxsize=None)
def _sc_gather_op(chunk, nbuf):
    nchunks = PER_W // chunk
    assert PER_W % chunk == 0 and nchunks % nbuf == 0
    mesh = plsc.VectorSubcoreMesh(core_axis_name="c", subcore_axis_name="s",
                                  num_cores=NC, num_subcores=NS)

    @functools.partial(
        pl.kernel,
        out_type=jax.ShapeDtypeStruct((E_PAD, D), jnp.float32),
        mesh=mesh,
        scratch_types=[
            pltpu.VMEM((PER_W,), jnp.int32),
            pltpu.VMEM((nbuf, chunk, D), jnp.float32),
        ] + [pltpu.SemaphoreType.DMA] * nbuf,
    )
    def gather_kernel(table, idx, out, idx_v, bufs, *sems):
        wid = lax.axis_index("s") * NC + lax.axis_index("c")
        base = wid * PER_W
        pltpu.sync_copy(idx.at[pl.ds(base, PER_W)], idx_v)

        def start(j, k):
            pltpu.async_copy(table.at[idx_v.at[pl.ds(j * chunk, chunk)]],
                             bufs.at[k], sems[k])

        def drain(k):
            pltpu.make_async_copy(table.at[idx_v.at[pl.ds(0, chunk)]],
                                  bufs.at[k], sems[k]).wait()

        for k in range(nbuf):
            start(k, k)

        def body(i, carry):
            for k in range(nbuf):
                j = i * nbuf + k
                drain(k)
                pltpu.sync_copy(bufs.at[k],
                                out.at[pl.ds(base + j * chunk, chunk)])
                start(j + nbuf, k)
            return carry

        lax.fori_loop(0, nchunks // nbuf - 1, body, 0)
        for k in range(nbuf):
            j = nchunks - nbuf + k
            drain(k)
            pltpu.sync_copy(bufs.at[k], out.at[pl.ds(base + j * chunk, chunk)])

    return gather_kernel


@functools.lru_cache(maxsize=None)
def _sc_scatter_op():
    mesh = plsc.VectorSubcoreMesh(core_axis_name="c", subcore_axis_name="s",
                                  num_cores=NC, num_subcores=NS)

    @functools.partial(
        pl.kernel,
        out_type=jax.ShapeDtypeStruct((NC * N_ACC, D), jnp.float32),
        mesh=mesh,
        scratch_types=[
            pltpu.VMEM((CHUNKS_PER_W, CHUNK), jnp.int32),
            pltpu.VMEM((NBUF_S, CHUNK, D), jnp.float32),
            pltpu.VMEM_SHARED((N_ACC, D), jnp.float32),
        ] + [pltpu.SemaphoreType.DMA] * NBUF_S,
    )
    def scatter_kernel(vals, idx3, zeros, out, idx_v, bufs, acc, *sems):
        cid = lax.axis_index("c")
        sid = lax.axis_index("s")
        row0 = sid * ACC_PER_TILE
        pltpu.sync_copy(zeros.at[pl.ds(row0, ACC_PER_TILE)],
                        acc.at[pl.ds(row0, ACC_PER_TILE)])
        wid = cid * NS + sid
        base = wid * PER_W
        pltpu.sync_copy(idx3.at[wid], idx_v)
        plsc.subcore_barrier()

        def start(j, k):
            pltpu.async_copy(vals.at[pl.ds(base + j * CHUNK, CHUNK)],
                             bufs.at[k], sems[k])

        def drain(k):
            pltpu.make_async_copy(vals.at[pl.ds(base, CHUNK)],
                                  bufs.at[k], sems[k]).wait()

        for k in range(NBUF_S):
            start(k, k)

        def body(i, carry):
            for k in range(NBUF_S):
                j = i * NBUF_S + k
                drain(k)
                pltpu.sync_copy(bufs.at[k], acc.at[idx_v.at[j]], add=True)
                start(j + NBUF_S, k)
            return carry

        lax.fori_loop(0, CHUNKS_PER_W // NBUF_S - 1, body, 0)
        for k in range(NBUF_S):
            j = CHUNKS_PER_W - NBUF_S + k
            drain(k)
            pltpu.sync_copy(bufs.at[k], acc.at[idx_v.at[j]], add=True)

        plsc.subcore_barrier()
        pltpu.sync_copy(acc.at[pl.ds(row0, ACC_PER_TILE)],
                        out.at[pl.ds(cid * N_ACC + row0, ACC_PER_TILE)])

    return scatter_kernel


@functools.lru_cache(maxsize=None)
def _sc_gather_spm_op():
    mesh = plsc.VectorSubcoreMesh(core_axis_name="c", subcore_axis_name="s",
                                  num_cores=NC, num_subcores=NS)

    @functools.partial(
        pl.kernel,
        out_type=jax.ShapeDtypeStruct((E_PAD, D), jnp.float32),
        mesh=mesh,
        scratch_types=[
            pltpu.VMEM((PER_W,), jnp.int32),
            pltpu.VMEM((NBUF_S, CHUNK, D), jnp.float32),
            pltpu.VMEM_SHARED((N_ACC, D), jnp.float32),
        ] + [pltpu.SemaphoreType.DMA] * NBUF_S,
    )
    def gather_spm_kernel(table, idx, out, idx_v, bufs, spm, *sems):
        cid = lax.axis_index("c")
        sid = lax.axis_index("s")
        row0 = sid * ACC_PER_TILE
        pltpu.sync_copy(table.at[pl.ds(row0, ACC_PER_TILE)],
                        spm.at[pl.ds(row0, ACC_PER_TILE)])
        wid = sid * NC + cid
        base = wid * PER_W
        pltpu.sync_copy(idx.at[pl.ds(base, PER_W)], idx_v)
        plsc.subcore_barrier()

        def start(j, k):
            pltpu.async_copy(spm.at[idx_v.at[pl.ds(j * CHUNK, CHUNK)]],
                             bufs.at[k], sems[k])

        def drain(k):
            pltpu.make_async_copy(spm.at[idx_v.at[pl.ds(0, CHUNK)]],
                                  bufs.at[k], sems[k]).wait()

        for k in range(NBUF_S):
            start(k, k)

        def body(i, carry):
            for k in range(NBUF_S):
                j = i * NBUF_S + k
                drain(k)
                pltpu.sync_copy(bufs.at[k],
                                out.at[pl.ds(base + j * CHUNK, CHUNK)])
                start(j + NBUF_S, k)
            return carry

        lax.fori_loop(0, CHUNKS_PER_W // NBUF_S - 1, body, 0)
        for k in range(NBUF_S):
            j = CHUNKS_PER_W - NBUF_S + k
            drain(k)
            pltpu.sync_copy(bufs.at[k], out.at[pl.ds(base + j * CHUNK, CHUNK)])

    return gather_spm_kernel


def _gather(table, idx):
    return _sc_gather_op(64, 8)(table, idx)


def _gather_spm(table, idx):
    return _sc_gather_spm_op()(table, idx)


def _scatter(vals, idx3, zeros):
    return _sc_scatter_op()(vals, idx3, zeros)


def _row_spec(r):
    return pl.BlockSpec((r, D), lambda i: (i, 0))


def _w_spec(r):
    return pl.BlockSpec((r, D), lambda i: (0, 0))


def _ln_relu(pre, lns, lno):
    x = jnp.maximum(pre, 0.0)
    mu = jnp.mean(x, axis=1, keepdims=True)
    xm = x - mu
    var = jnp.mean(xm * xm, axis=1, keepdims=True)
    return xm * lax.rsqrt(var + 1e-5) * lns + lno


def _dot(a, b):
    return jnp.dot(a, b, preferred_element_type=jnp.float32)


def _edge0_body(sf_ref, ed_ref, g_ref, was_ref, wae_ref, wbs_ref, b1_ref,
                w2_ref, w1e_ref, b2_ref, lns_ref, lno_ref,
                e_ref, bn_ref, bb_ref):
    sf = sf_ref[...]
    b1 = b1_ref[...]
    ba = _dot(ed_ref[...], wae_ref[...]) + _dot(sf, was_ref[...]) + b1
    bb = _dot(sf, wbs_ref[...]) + b1
    bb_ref[...] = bb
    h = _ln_relu(ba + g_ref[...], lns_ref[...], lno_ref[...])
    e = _dot(h, w2_ref[...]) + b2_ref[...]
    e_ref[...] = e
    bn_ref[...] = bb + _dot(e, w1e_ref[...])


def _edge0(s_feat, edges8, g, w1as, w1ae8, w1bs, b1r, w2, w1e, b2r, lnsr, lnor):
    return pl.pallas_call(
        _edge0_body,
        grid=(E_PAD // BE,),
        in_specs=[_row_spec(BE), pl.BlockSpec((BE, 8), lambda i: (i, 0)),
                  _row_spec(BE),
                  _w_spec(D), pl.BlockSpec((8, D), lambda i: (0, 0)),
                  _w_spec(D), _w_spec(1),
                  _w_spec(D), _w_spec(D), _w_spec(1), _w_spec(1), _w_spec(1)],
        out_specs=[_row_spec(BE), _row_spec(BE), _row_spec(BE)],
        out_shape=[jax.ShapeDtypeStruct((E_PAD, D), jnp.float32)] * 3,
    )(s_feat, edges8, g, w1as, w1ae8, w1bs, b1r, w2, w1e, b2r, lnsr, lnor)


def _proj_body(x_ref, w_ref, o_ref):
    o_ref[...] = _dot(x_ref[...], w_ref[...])


def _proj(x, w):
    r = x.shape[0]
    return pl.pallas_call(
        _proj_body,
        grid=(r // BN,),
        in_specs=[_row_spec(BN), _w_spec(D)],
        out_specs=_row_spec(BN),
        out_shape=jax.ShapeDtypeStruct((r, D), jnp.float32),
    )(x, w)


def _edge_full_body(base_ref, bb_ref, g_ref, w2_ref, w1e_ref, b2_ref,
                    lns_ref, lno_ref, e_ref, bn_ref):
    h = _ln_relu(base_ref[...] + g_ref[...], lns_ref[...], lno_ref[...])
    e = _dot(h, w2_ref[...]) + b2_ref[...]
    e_ref[...] = e
    bn_ref[...] = bb_ref[...] + _dot(e, w1e_ref[...])


def _edge_full(base, bb, g, w2, w1e, b2r, lnsr, lnor):
    return pl.pallas_call(
        _edge_full_body,
        grid=(E_PAD // BE,),
        in_specs=[_row_spec(BE), _row_spec(BE), _row_spec(BE),
                  _w_spec(D), _w_spec(D), _w_spec(1), _w_spec(1), _w_spec(1)],
        out_specs=[_row_spec(BE), _row_spec(BE)],
        out_shape=[jax.ShapeDtypeStruct((E_PAD, D), jnp.float32)] * 2,
    )(base, bb, g, w2, w1e, b2r, lnsr, lnor)


def _edge_last_body(base_ref, g_ref, w2_ref, b2_ref, lns_ref, lno_ref, e_ref):
    h = _ln_relu(base_ref[...] + g_ref[...], lns_ref[...], lno_ref[...])
    e_ref[...] = _dot(h, w2_ref[...]) + b2_ref[...]


def _edge_last(base, g, w2, b2r, lnsr, lnor):
    return pl.pallas_call(
        _edge_last_body,
        grid=(E_PAD // BE,),
        in_specs=[_row_spec(BE), _row_spec(BE),
                  _w_spec(D), _w_spec(1), _w_spec(1), _w_spec(1)],
        out_specs=_row_spec(BE),
        out_shape=jax.ShapeDtypeStruct((E_PAD, D), jnp.float32),
    )(base, g, w2, b2r, lnsr, lnor)


def _node_body(sph_ref, m0_ref, m1_ref, w1t_ref, w1b_ref, b1_ref,
               lns_ref, lno_ref, w2_ref, b2_ref, wr_ref, out_ref, p_ref):
    msg = m0_ref[...] + m1_ref[...]
    pre = _dot(sph_ref[...], w1t_ref[...]) + _dot(msg, w1b_ref[...]) + b1_ref[...]
    h = _ln_relu(pre, lns_ref[...], lno_ref[...])
    upd = _dot(h, w2_ref[...]) + b2_ref[...]
    out_ref[...] = upd
    p_ref[...] = _dot(upd, wr_ref[...])


def _node(sphere, acc, w1t, w1b, b1r, lnsr, lnor, w2, b2r, wr):
    nblk = N_ACC // BN
    return pl.pallas_call(
        _node_body,
        grid=(nblk,),
        in_specs=[_row_spec(BN),
                  pl.BlockSpec((BN, D), lambda i: (i, 0)),
                  pl.BlockSpec((BN, D), lambda i: (i + N_ACC // BN, 0)),
                  _w_spec(D), _w_spec(D), _w_spec(1), _w_spec(1), _w_spec(1),
                  _w_spec(D), _w_spec(1), _w_spec(D)],
        out_specs=[_row_spec(BN), _row_spec(BN)],
        out_shape=[jax.ShapeDtypeStruct((N_ACC, D), jnp.float32)] * 2,
    )(sphere, acc, acc, w1t, w1b, b1r, lnsr, lnor, w2, b2r, wr)


def kernel(spatial_nodes, sphere_nodes, edges, senders, receivers,
           edge_W1a, edge_W1b, edge_b1, edge_ln_s, edge_ln_o, edge_W2, edge_b2,
           node_W1, node_b1, node_ln_s, node_ln_o, node_W2, node_b2):
    f32 = jnp.float32
    w1a_e, w1a_s, w1a_r = edge_W1a[:3], edge_W1a[3:3 + D], edge_W1a[3 + D:]
    w1b_e, w1b_s, w1b_r = edge_W1b[:D], edge_W1b[D:2 * D], edge_W1b[2 * D:]
    w1ae8 = jnp.zeros((8, D), f32).at[:3].set(w1a_e)
    nw1_t, nw1_b = node_W1[:D], node_W1[D:]

    edges8 = jnp.zeros((E_PAD, 8), f32).at[:E, :3].set(edges)
    send_pad = jnp.zeros((E_PAD,), jnp.int32).at[:E].set(senders)
    dummy_tgt = DUMMY + jnp.arange(E_PAD, dtype=jnp.int32) % (N_ACC - N_SPHERE)
    recv_pad = dummy_tgt.at[:E].set(receivers)
    recv3 = recv_pad.reshape(NW, CHUNKS_PER_W, CHUNK)
    sphere_pad = jnp.zeros((N_ACC, D), f32).at[:N_SPHERE].set(sphere_nodes)
    zeros_acc = jnp.zeros((N_ACC, D), f32)

    b1r = edge_b1.reshape(1, D)
    b2r = edge_b2.reshape(1, D)
    lnsr = edge_ln_s.reshape(1, D)
    lnor = edge_ln_o.reshape(1, D)
    nb1r = node_b1.reshape(1, D)
    nb2r = node_b2.reshape(1, D)
    nlnsr = node_ln_s.reshape(1, D)
    nlnor = node_ln_o.reshape(1, D)

    s_feat = _gather(spatial_nodes, send_pad)
    proj = _proj(sphere_pad, w1a_r)
    sphere = sphere_pad
    base = bb = None
    for t in range(STEPS_):
        g = _gather_spm(proj, recv_pad)
        if t == 0:
            e_out, base, bb = _edge0(s_feat, edges8, g, w1a_s, w1ae8, w1b_s,
                                     b1r, edge_W2, w1b_e, b2r, lnsr, lnor)
        elif t < STEPS_ - 1:
            e_out, base = _edge_full(base, bb, g, edge_W2, w1b_e,
                                     b2r, lnsr, lnor)
        else:
            e_out = _edge_last(base, g, edge_W2, b2r, lnsr, lnor)
        acc = _scatter(e_out, recv3, zeros_acc)
        sphere, proj = _node(sphere, acc, nw1_t, nw1_b, nb1r, nlnsr, nlnor,
                             node_W2, nb2r, w1b_r)
    return sphere[:N_SPHERE]


# speedup vs baseline: 3.9456x; 1.0114x over previous
import functools

import jax
import jax.numpy as jnp
from jax import lax
from jax.experimental import pallas as pl
from jax.experimental.pallas import tpu as pltpu
from jax.experimental.pallas import tpu_sc as plsc

N_SPATIAL = 65160
N_SPHERE = 10242
E = 61452
D = 128
STEPS_ = 3

NC = 2
NS = 16
NW = NC * NS

CHUNK = 128
CHUNKS_PER_W = 16
PER_W = CHUNK * CHUNKS_PER_W
E_PAD = PER_W * NW

N_ACC = 10368
ACC_PER_TILE = N_ACC // NS
DUMMY = N_SPHERE

BE = 4096
BN = 1152


NBUF = 4
NBUF_S = 2


@functools.lru_cache(maxsize=None)
def _sc_gather_op(chunk, nbuf):
    nchunks = PER_W // chunk
    assert PER_W % chunk == 0 and nchunks % nbuf == 0
    mesh = plsc.VectorSubcoreMesh(core_axis_name="c", subcore_axis_name="s",
                                  num_cores=NC, num_subcores=NS)

    @functools.partial(
        pl.kernel,
        out_type=jax.ShapeDtypeStruct((E_PAD, D), jnp.float32),
        mesh=mesh,
        scratch_types=[
            pltpu.VMEM((PER_W,), jnp.int32),
            pltpu.VMEM((nbuf, chunk, D), jnp.float32),
        ] + [pltpu.SemaphoreType.DMA] * nbuf,
    )
    def gather_kernel(table, idx, out, idx_v, bufs, *sems):
        wid = lax.axis_index("s") * NC + lax.axis_index("c")
        base = wid * PER_W
        pltpu.sync_copy(idx.at[pl.ds(base, PER_W)], idx_v)

        def start(j, k):
            pltpu.async_copy(table.at[idx_v.at[pl.ds(j * chunk, chunk)]],
                             bufs.at[k], sems[k])

        def drain(k):
            pltpu.make_async_copy(table.at[idx_v.at[pl.ds(0, chunk)]],
                                  bufs.at[k], sems[k]).wait()

        for k in range(nbuf):
            start(k, k)

        def body(i, carry):
            for k in range(nbuf):
                j = i * nbuf + k
                drain(k)
                pltpu.sync_copy(bufs.at[k],
                                out.at[pl.ds(base + j * chunk, chunk)])
                start(j + nbuf, k)
            return carry

        lax.fori_loop(0, nchunks // nbuf - 1, body, 0)
        for k in range(nbuf):
            j = nchunks - nbuf + k
            drain(k)
            pltpu.sync_copy(bufs.at[k], out.at[pl.ds(base + j * chunk, chunk)])

    return gather_kernel


@functools.lru_cache(maxsize=None)
def _sc_scatter_op(chunk, nbuf):
    nchunks = PER_W // chunk
    assert PER_W % chunk == 0 and nchunks % nbuf == 0
    mesh = plsc.VectorSubcoreMesh(core_axis_name="c", subcore_axis_name="s",
                                  num_cores=NC, num_subcores=NS)

    @functools.partial(
        pl.kernel,
        out_type=jax.ShapeDtypeStruct((NC * N_ACC, D), jnp.float32),
        mesh=mesh,
        scratch_types=[
            pltpu.VMEM((nchunks, chunk), jnp.int32),
            pltpu.VMEM((nbuf, chunk, D), jnp.float32),
            pltpu.VMEM_SHARED((N_ACC, D), jnp.float32),
        ] + [pltpu.SemaphoreType.DMA] * nbuf,
    )
    def scatter_kernel(vals, idx3, zeros, out, idx_v, bufs, acc, *sems):
        cid = lax.axis_index("c")
        sid = lax.axis_index("s")
        row0 = sid * ACC_PER_TILE
        pltpu.sync_copy(zeros.at[pl.ds(row0, ACC_PER_TILE)],
                        acc.at[pl.ds(row0, ACC_PER_TILE)])
        wid = cid * NS + sid
        base = wid * PER_W
        pltpu.sync_copy(idx3.at[wid], idx_v)
        plsc.subcore_barrier()

        def start(j, k):
            pltpu.async_copy(vals.at[pl.ds(base + j * chunk, chunk)],
                             bufs.at[k], sems[k])

        def drain(k):
            pltpu.make_async_copy(vals.at[pl.ds(base, chunk)],
                                  bufs.at[k], sems[k]).wait()

        for k in range(nbuf):
            start(k, k)

        def body(i, carry):
            for k in range(nbuf):
                j = i * nbuf + k
                drain(k)
                pltpu.sync_copy(bufs.at[k], acc.at[idx_v.at[j]], add=True)
                start(j + nbuf, k)
            return carry

        lax.fori_loop(0, nchunks // nbuf - 1, body, 0)
        for k in range(nbuf):
            j = nchunks - nbuf + k
            drain(k)
            pltpu.sync_copy(bufs.at[k], acc.at[idx_v.at[j]], add=True)

        plsc.subcore_barrier()
        pltpu.sync_copy(acc.at[pl.ds(row0, ACC_PER_TILE)],
                        out.at[pl.ds(cid * N_ACC + row0, ACC_PER_TILE)])

    return scatter_kernel


@functools.lru_cache(maxsize=None)
def _sc_gather_spm_op(chunk, nbuf):
    nchunks = PER_W // chunk
    assert PER_W % chunk == 0 and nchunks % nbuf == 0
    mesh = plsc.VectorSubcoreMesh(core_axis_name="c", subcore_axis_name="s",
                                  num_cores=NC, num_subcores=NS)

    @functools.partial(
        pl.kernel,
        out_type=jax.ShapeDtypeStruct((E_PAD, D), jnp.float32),
        mesh=mesh,
        scratch_types=[
            pltpu.VMEM((PER_W,), jnp.int32),
            pltpu.VMEM((nbuf, chunk, D), jnp.float32),
            pltpu.VMEM_SHARED((N_ACC, D), jnp.float32),
        ] + [pltpu.SemaphoreType.DMA] * nbuf,
    )
    def gather_spm_kernel(table, idx, out, idx_v, bufs, spm, *sems):
        cid = lax.axis_index("c")
        sid = lax.axis_index("s")
        row0 = sid * ACC_PER_TILE
        pltpu.sync_copy(table.at[pl.ds(row0, ACC_PER_TILE)],
                        spm.at[pl.ds(row0, ACC_PER_TILE)])
        wid = sid * NC + cid
        base = wid * PER_W
        pltpu.sync_copy(idx.at[pl.ds(base, PER_W)], idx_v)
        plsc.subcore_barrier()

        def start(j, k):
            pltpu.async_copy(spm.at[idx_v.at[pl.ds(j * chunk, chunk)]],
                             bufs.at[k], sems[k])

        def drain(k):
            pltpu.make_async_copy(spm.at[idx_v.at[pl.ds(0, chunk)]],
                                  bufs.at[k], sems[k]).wait()

        for k in range(nbuf):
            start(k, k)

        def body(i, carry):
            for k in range(nbuf):
                j = i * nbuf + k
                drain(k)
                pltpu.sync_copy(bufs.at[k],
                                out.at[pl.ds(base + j * chunk, chunk)])
                start(j + nbuf, k)
            return carry

        lax.fori_loop(0, nchunks // nbuf - 1, body, 0)
        for k in range(nbuf):
            j = nchunks - nbuf + k
            drain(k)
            pltpu.sync_copy(bufs.at[k], out.at[pl.ds(base + j * chunk, chunk)])

    return gather_spm_kernel


def _gather(table, idx):
    return _sc_gather_op(64, 8)(table, idx)


def _gather_spm(table, idx):
    return _sc_gather_spm_op(64, 4)(table, idx)


SC_CHUNK = 64
SC_NBUF = 4


def _scatter(vals, idx3, zeros):
    return _sc_scatter_op(SC_CHUNK, SC_NBUF)(vals, idx3, zeros)


def _row_spec(r):
    return pl.BlockSpec((r, D), lambda i: (i, 0))


def _w_spec(r):
    return pl.BlockSpec((r, D), lambda i: (0, 0))


def _ln_relu(pre, lns, lno):
    x = jnp.maximum(pre, 0.0)
    mu = jnp.mean(x, axis=1, keepdims=True)
    xm = x - mu
    var = jnp.mean(xm * xm, axis=1, keepdims=True)
    return xm * lax.rsqrt(var + 1e-5) * lns + lno


def _dot(a, b):
    return jnp.dot(a, b, preferred_element_type=jnp.float32)


def _edge0_body(sf_ref, ed_ref, g_ref, was_ref, wae_ref, wbs_ref, b1_ref,
                w2_ref, w1e_ref, b2_ref, lns_ref, lno_ref,
                e_ref, bn_ref, bb_ref):
    sf = sf_ref[...]
    b1 = b1_ref[...]
    ba = _dot(ed_ref[...], wae_ref[...]) + _dot(sf, was_ref[...]) + b1
    bb = _dot(sf, wbs_ref[...]) + b1
    bb_ref[...] = bb
    h = _ln_relu(ba + g_ref[...], lns_ref[...], lno_ref[...])
    e = _dot(h, w2_ref[...]) + b2_ref[...]
    e_ref[...] = e
    bn_ref[...] = bb + _dot(e, w1e_ref[...])


def _edge0(s_feat, edges8, g, w1as, w1ae8, w1bs, b1r, w2, w1e, b2r, lnsr, lnor):
    return pl.pallas_call(
        _edge0_body,
        grid=(E_PAD // BE,),
        in_specs=[_row_spec(BE), pl.BlockSpec((BE, 8), lambda i: (i, 0)),
                  _row_spec(BE),
                  _w_spec(D), pl.BlockSpec((8, D), lambda i: (0, 0)),
                  _w_spec(D), _w_spec(1),
                  _w_spec(D), _w_spec(D), _w_spec(1), _w_spec(1), _w_spec(1)],
        out_specs=[_row_spec(BE), _row_spec(BE), _row_spec(BE)],
        out_shape=[jax.ShapeDtypeStruct((E_PAD, D), jnp.float32)] * 3,
    )(s_feat, edges8, g, w1as, w1ae8, w1bs, b1r, w2, w1e, b2r, lnsr, lnor)


def _proj_body(x_ref, w_ref, o_ref):
    o_ref[...] = _dot(x_ref[...], w_ref[...])


def _proj(x, w):
    r = x.shape[0]
    return pl.pallas_call(
        _proj_body,
        grid=(r // BN,),
        in_specs=[_row_spec(BN), _w_spec(D)],
        out_specs=_row_spec(BN),
        out_shape=jax.ShapeDtypeStruct((r, D), jnp.float32),
    )(x, w)


def _edge_full_body(base_ref, bb_ref, g_ref, w2_ref, w1e_ref, b2_ref,
                    lns_ref, lno_ref, e_ref, bn_ref):
    h = _ln_relu(base_ref[...] + g_ref[...], lns_ref[...], lno_ref[...])
    e = _dot(h, w2_ref[...]) + b2_ref[...]
    e_ref[...] = e
    bn_ref[...] = bb_ref[...] + _dot(e, w1e_ref[...])


def _edge_full(base, bb, g, w2, w1e, b2r, lnsr, lnor):
    return pl.pallas_call(
        _edge_full_body,
        grid=(E_PAD // BE,),
        in_specs=[_row_spec(BE), _row_spec(BE), _row_spec(BE),
                  _w_spec(D), _w_spec(D), _w_spec(1), _w_spec(1), _w_spec(1)],
        out_specs=[_row_spec(BE), _row_spec(BE)],
        out_shape=[jax.ShapeDtypeStruct((E_PAD, D), jnp.float32)] * 2,
    )(base, bb, g, w2, w1e, b2r, lnsr, lnor)


def _edge_last_body(base_ref, g_ref, w2_ref, b2_ref, lns_ref, lno_ref, e_ref):
    h = _ln_relu(base_ref[...] + g_ref[...], lns_ref[...], lno_ref[...])
    e_ref[...] = _dot(h, w2_ref[...]) + b2_ref[...]


def _edge_last(base, g, w2, b2r, lnsr, lnor):
    return pl.pallas_call(
        _edge_last_body,
        grid=(E_PAD // BE,),
        in_specs=[_row_spec(BE), _row_spec(BE),
                  _w_spec(D), _w_spec(1), _w_spec(1), _w_spec(1)],
        out_specs=_row_spec(BE),
        out_shape=jax.ShapeDtypeStruct((E_PAD, D), jnp.float32),
    )(base, g, w2, b2r, lnsr, lnor)


def _node_body(sph_ref, m0_ref, m1_ref, w1t_ref, w1b_ref, b1_ref,
               lns_ref, lno_ref, w2_ref, b2_ref, wr_ref, out_ref, p_ref):
    msg = m0_ref[...] + m1_ref[...]
    pre = _dot(sph_ref[...], w1t_ref[...]) + _dot(msg, w1b_ref[...]) + b1_ref[...]
    h = _ln_relu(pre, lns_ref[...], lno_ref[...])
    upd = _dot(h, w2_ref[...]) + b2_ref[...]
    out_ref[...] = upd
    p_ref[...] = _dot(upd, wr_ref[...])


def _node(sphere, acc, w1t, w1b, b1r, lnsr, lnor, w2, b2r, wr):
    nblk = N_ACC // BN
    return pl.pallas_call(
        _node_body,
        grid=(nblk,),
        in_specs=[_row_spec(BN),
                  pl.BlockSpec((BN, D), lambda i: (i, 0)),
                  pl.BlockSpec((BN, D), lambda i: (i + N_ACC // BN, 0)),
                  _w_spec(D), _w_spec(D), _w_spec(1), _w_spec(1), _w_spec(1),
                  _w_spec(D), _w_spec(1), _w_spec(D)],
        out_specs=[_row_spec(BN), _row_spec(BN)],
        out_shape=[jax.ShapeDtypeStruct((N_ACC, D), jnp.float32)] * 2,
    )(sphere, acc, acc, w1t, w1b, b1r, lnsr, lnor, w2, b2r, wr)


def kernel(spatial_nodes, sphere_nodes, edges, senders, receivers,
           edge_W1a, edge_W1b, edge_b1, edge_ln_s, edge_ln_o, edge_W2, edge_b2,
           node_W1, node_b1, node_ln_s, node_ln_o, node_W2, node_b2):
    f32 = jnp.float32
    w1a_e, w1a_s, w1a_r = edge_W1a[:3], edge_W1a[3:3 + D], edge_W1a[3 + D:]
    w1b_e, w1b_s, w1b_r = edge_W1b[:D], edge_W1b[D:2 * D], edge_W1b[2 * D:]
    w1ae8 = jnp.zeros((8, D), f32).at[:3].set(w1a_e)
    nw1_t, nw1_b = node_W1[:D], node_W1[D:]

    edges8 = jnp.zeros((E_PAD, 8), f32).at[:E, :3].set(edges)
    send_pad = jnp.zeros((E_PAD,), jnp.int32).at[:E].set(senders)
    dummy_tgt = DUMMY + jnp.arange(E_PAD, dtype=jnp.int32) % (N_ACC - N_SPHERE)
    recv_pad = dummy_tgt.at[:E].set(receivers)
    recv3 = recv_pad.reshape(NW, PER_W // SC_CHUNK, SC_CHUNK)
    sphere_pad = jnp.zeros((N_ACC, D), f32).at[:N_SPHERE].set(sphere_nodes)
    zeros_acc = jnp.zeros((N_ACC, D), f32)

    b1r = edge_b1.reshape(1, D)
    b2r = edge_b2.reshape(1, D)
    lnsr = edge_ln_s.reshape(1, D)
    lnor = edge_ln_o.reshape(1, D)
    nb1r = node_b1.reshape(1, D)
    nb2r = node_b2.reshape(1, D)
    nlnsr = node_ln_s.reshape(1, D)
    nlnor = node_ln_o.reshape(1, D)

    s_feat = _gather(spatial_nodes, send_pad)
    proj = _proj(sphere_pad, w1a_r)
    sphere = sphere_pad
    base = bb = None
    for t in range(STEPS_):
        g = _gather_spm(proj, recv_pad)
        if t == 0:
            e_out, base, bb = _edge0(s_feat, edges8, g, w1a_s, w1ae8, w1b_s,
                                     b1r, edge_W2, w1b_e, b2r, lnsr, lnor)
        elif t < STEPS_ - 1:
            e_out, base = _edge_full(base, bb, g, edge_W2, w1b_e,
                                     b2r, lnsr, lnor)
        else:
            e_out = _edge_last(base, g, edge_W2, b2r, lnsr, lnor)
        acc = _scatter(e_out, recv3, zeros_acc)
        sphere, proj = _node(sphere, acc, nw1_t, nw1_b, nb1r, nlnsr, nlnor,
                             node_W2, nb2r, w1b_r)
    return sphere[:N_SPHERE]


# speedup vs baseline: 3.9766x; 1.0079x over previous
import functools

import jax
import jax.numpy as jnp
from jax import lax
from jax.experimental import pallas as pl
from jax.experimental.pallas import tpu as pltpu
from jax.experimental.pallas import tpu_sc as plsc

N_SPATIAL = 65160
N_SPHERE = 10242
E = 61452
D = 128
STEPS_ = 3

NC = 2
NS = 16
NW = NC * NS

CHUNK = 128
CHUNKS_PER_W = 16
PER_W = CHUNK * CHUNKS_PER_W
E_PAD = PER_W * NW

N_ACC = 10368
ACC_PER_TILE = N_ACC // NS
DUMMY = N_SPHERE

BE = 8192
BN = 1152


NBUF = 4
NBUF_S = 2


@functools.lru_cache(maxsize=None)
def _sc_gather_op(chunk, nbuf):
    nchunks = PER_W // chunk
    assert PER_W % chunk == 0 and nchunks % nbuf == 0
    mesh = plsc.VectorSubcoreMesh(core_axis_name="c", subcore_axis_name="s",
                                  num_cores=NC, num_subcores=NS)

    @functools.partial(
        pl.kernel,
        out_type=jax.ShapeDtypeStruct((E_PAD, D), jnp.float32),
        mesh=mesh,
        scratch_types=[
            pltpu.VMEM((PER_W,), jnp.int32),
            pltpu.VMEM((nbuf, chunk, D), jnp.float32),
        ] + [pltpu.SemaphoreType.DMA] * nbuf,
    )
    def gather_kernel(table, idx, out, idx_v, bufs, *sems):
        wid = lax.axis_index("s") * NC + lax.axis_index("c")
        base = wid * PER_W
        pltpu.sync_copy(idx.at[pl.ds(base, PER_W)], idx_v)

        def start(j, k):
            pltpu.async_copy(table.at[idx_v.at[pl.ds(j * chunk, chunk)]],
                             bufs.at[k], sems[k])

        def drain(k):
            pltpu.make_async_copy(table.at[idx_v.at[pl.ds(0, chunk)]],
                                  bufs.at[k], sems[k]).wait()

        for k in range(nbuf):
            start(k, k)

        def body(i, carry):
            for k in range(nbuf):
                j = i * nbuf + k
                drain(k)
                pltpu.sync_copy(bufs.at[k],
                                out.at[pl.ds(base + j * chunk, chunk)])
                start(j + nbuf, k)
            return carry

        lax.fori_loop(0, nchunks // nbuf - 1, body, 0)
        for k in range(nbuf):
            j = nchunks - nbuf + k
            drain(k)
            pltpu.sync_copy(bufs.at[k], out.at[pl.ds(base + j * chunk, chunk)])

    return gather_kernel


@functools.lru_cache(maxsize=None)
def _sc_scatter_op(chunk, nbuf):
    nchunks = PER_W // chunk
    assert PER_W % chunk == 0 and nchunks % nbuf == 0
    mesh = plsc.VectorSubcoreMesh(core_axis_name="c", subcore_axis_name="s",
                                  num_cores=NC, num_subcores=NS)

    @functools.partial(
        pl.kernel,
        out_type=jax.ShapeDtypeStruct((NC * N_ACC, D), jnp.float32),
        mesh=mesh,
        scratch_types=[
            pltpu.VMEM((nchunks, chunk), jnp.int32),
            pltpu.VMEM((nbuf, chunk, D), jnp.float32),
            pltpu.VMEM_SHARED((N_ACC, D), jnp.float32),
        ] + [pltpu.SemaphoreType.DMA] * nbuf,
    )
    def scatter_kernel(vals, idx3, zeros, out, idx_v, bufs, acc, *sems):
        cid = lax.axis_index("c")
        sid = lax.axis_index("s")
        row0 = sid * ACC_PER_TILE
        pltpu.sync_copy(zeros.at[pl.ds(row0, ACC_PER_TILE)],
                        acc.at[pl.ds(row0, ACC_PER_TILE)])
        wid = cid * NS + sid
        base = wid * PER_W
        pltpu.sync_copy(idx3.at[wid], idx_v)
        plsc.subcore_barrier()

        def start(j, k):
            pltpu.async_copy(vals.at[pl.ds(base + j * chunk, chunk)],
                             bufs.at[k], sems[k])

        def drain(k):
            pltpu.make_async_copy(vals.at[pl.ds(base, chunk)],
                                  bufs.at[k], sems[k]).wait()

        for k in range(nbuf):
            start(k, k)

        def body(i, carry):
            for k in range(nbuf):
                j = i * nbuf + k
                drain(k)
                pltpu.sync_copy(bufs.at[k], acc.at[idx_v.at[j]], add=True)
                start(j + nbuf, k)
            return carry

        lax.fori_loop(0, nchunks // nbuf - 1, body, 0)
        for k in range(nbuf):
            j = nchunks - nbuf + k
            drain(k)
            pltpu.sync_copy(bufs.at[k], acc.at[idx_v.at[j]], add=True)

        plsc.subcore_barrier()
        pltpu.sync_copy(acc.at[pl.ds(row0, ACC_PER_TILE)],
                        out.at[pl.ds(cid * N_ACC + row0, ACC_PER_TILE)])

    return scatter_kernel


@functools.lru_cache(maxsize=None)
def _sc_gather_spm_op(chunk, nbuf):
    nchunks = PER_W // chunk
    assert PER_W % chunk == 0 and nchunks % nbuf == 0
    mesh = plsc.VectorSubcoreMesh(core_axis_name="c", subcore_axis_name="s",
                                  num_cores=NC, num_subcores=NS)

    @functools.partial(
        pl.kernel,
        out_type=jax.ShapeDtypeStruct((E_PAD, D), jnp.float32),
        mesh=mesh,
        scratch_types=[
            pltpu.VMEM((PER_W,), jnp.int32),
            pltpu.VMEM((nbuf, chunk, D), jnp.float32),
            pltpu.VMEM_SHARED((N_ACC, D), jnp.float32),
        ] + [pltpu.SemaphoreType.DMA] * nbuf,
    )
    def gather_spm_kernel(table, idx, out, idx_v, bufs, spm, *sems):
        cid = lax.axis_index("c")
        sid = lax.axis_index("s")
        row0 = sid * ACC_PER_TILE
        pltpu.sync_copy(table.at[pl.ds(row0, ACC_PER_TILE)],
                        spm.at[pl.ds(row0, ACC_PER_TILE)])
        wid = sid * NC + cid
        base = wid * PER_W
        pltpu.sync_copy(idx.at[pl.ds(base, PER_W)], idx_v)
        plsc.subcore_barrier()

        def start(j, k):
            pltpu.async_copy(spm.at[idx_v.at[pl.ds(j * chunk, chunk)]],
                             bufs.at[k], sems[k])

        def drain(k):
            pltpu.make_async_copy(spm.at[idx_v.at[pl.ds(0, chunk)]],
                                  bufs.at[k], sems[k]).wait()

        for k in range(nbuf):
            start(k, k)

        def body(i, carry):
            for k in range(nbuf):
                j = i * nbuf + k
                drain(k)
                pltpu.sync_copy(bufs.at[k],
                                out.at[pl.ds(base + j * chunk, chunk)])
                start(j + nbuf, k)
            return carry

        lax.fori_loop(0, nchunks // nbuf - 1, body, 0)
        for k in range(nbuf):
            j = nchunks - nbuf + k
            drain(k)
            pltpu.sync_copy(bufs.at[k], out.at[pl.ds(base + j * chunk, chunk)])

    return gather_spm_kernel


def _gather(table, idx):
    return _sc_gather_op(64, 8)(table, idx)


def _gather_spm(table, idx):
    return _sc_gather_spm_op(64, 4)(table, idx)


SC_CHUNK = 64
SC_NBUF = 4


def _scatter(vals, idx3, zeros):
    return _sc_scatter_op(SC_CHUNK, SC_NBUF)(vals, idx3, zeros)


def _row_spec(r):
    return pl.BlockSpec((r, D), lambda i: (i, 0))


def _w_spec(r):
    return pl.BlockSpec((r, D), lambda i: (0, 0))


def _ln_relu(pre, lns, lno):
    x = jnp.maximum(pre, 0.0)
    mu = jnp.mean(x, axis=1, keepdims=True)
    xm = x - mu
    var = jnp.mean(xm * xm, axis=1, keepdims=True)
    return xm * lax.rsqrt(var + 1e-5) * lns + lno


def _dot(a, b):
    return jnp.dot(a, b, preferred_element_type=jnp.float32)


def _edge0_body(sf_ref, ed_ref, g_ref, was_ref, wae_ref, wbs_ref, b1_ref,
                w2_ref, w1e_ref, b2_ref, lns_ref, lno_ref,
                e_ref, bn_ref, bb_ref):
    sf = sf_ref[...]
    b1 = b1_ref[...]
    ba = _dot(ed_ref[...], wae_ref[...]) + _dot(sf, was_ref[...]) + b1
    bb = _dot(sf, wbs_ref[...]) + b1
    bb_ref[...] = bb
    h = _ln_relu(ba + g_ref[...], lns_ref[...], lno_ref[...])
    e = _dot(h, w2_ref[...]) + b2_ref[...]
    e_ref[...] = e
    bn_ref[...] = bb + _dot(e, w1e_ref[...])


def _edge0(s_feat, edges8, g, w1as, w1ae8, w1bs, b1r, w2, w1e, b2r, lnsr, lnor):
    return pl.pallas_call(
        _edge0_body,
        grid=(E_PAD // BE,),
        in_specs=[_row_spec(BE), pl.BlockSpec((BE, 8), lambda i: (i, 0)),
                  _row_spec(BE),
                  _w_spec(D), pl.BlockSpec((8, D), lambda i: (0, 0)),
                  _w_spec(D), _w_spec(1),
                  _w_spec(D), _w_spec(D), _w_spec(1), _w_spec(1), _w_spec(1)],
        out_specs=[_row_spec(BE), _row_spec(BE), _row_spec(BE)],
        out_shape=[jax.ShapeDtypeStruct((E_PAD, D), jnp.float32)] * 3,
    )(s_feat, edges8, g, w1as, w1ae8, w1bs, b1r, w2, w1e, b2r, lnsr, lnor)


def _proj_body(x_ref, w_ref, o_ref):
    o_ref[...] = _dot(x_ref[...], w_ref[...])


def _proj(x, w):
    r = x.shape[0]
    return pl.pallas_call(
        _proj_body,
        grid=(r // BN,),
        in_specs=[_row_spec(BN), _w_spec(D)],
        out_specs=_row_spec(BN),
        out_shape=jax.ShapeDtypeStruct((r, D), jnp.float32),
    )(x, w)


def _edge_full_body(base_ref, bb_ref, g_ref, w2_ref, w1e_ref, b2_ref,
                    lns_ref, lno_ref, e_ref, bn_ref):
    h = _ln_relu(base_ref[...] + g_ref[...], lns_ref[...], lno_ref[...])
    e = _dot(h, w2_ref[...]) + b2_ref[...]
    e_ref[...] = e
    bn_ref[...] = bb_ref[...] + _dot(e, w1e_ref[...])


def _edge_full(base, bb, g, w2, w1e, b2r, lnsr, lnor):
    return pl.pallas_call(
        _edge_full_body,
        grid=(E_PAD // BE,),
        in_specs=[_row_spec(BE), _row_spec(BE), _row_spec(BE),
                  _w_spec(D), _w_spec(D), _w_spec(1), _w_spec(1), _w_spec(1)],
        out_specs=[_row_spec(BE), _row_spec(BE)],
        out_shape=[jax.ShapeDtypeStruct((E_PAD, D), jnp.float32)] * 2,
    )(base, bb, g, w2, w1e, b2r, lnsr, lnor)


def _edge_last_body(base_ref, g_ref, w2_ref, b2_ref, lns_ref, lno_ref, e_ref):
    h = _ln_relu(base_ref[...] + g_ref[...], lns_ref[...], lno_ref[...])
    e_ref[...] = _dot(h, w2_ref[...]) + b2_ref[...]


def _edge_last(base, g, w2, b2r, lnsr, lnor):
    return pl.pallas_call(
        _edge_last_body,
        grid=(E_PAD // BE,),
        in_specs=[_row_spec(BE), _row_spec(BE),
                  _w_spec(D), _w_spec(1), _w_spec(1), _w_spec(1)],
        out_specs=_row_spec(BE),
        out_shape=jax.ShapeDtypeStruct((E_PAD, D), jnp.float32),
    )(base, g, w2, b2r, lnsr, lnor)


def _node_body(sph_ref, m0_ref, m1_ref, w1t_ref, w1b_ref, b1_ref,
               lns_ref, lno_ref, w2_ref, b2_ref, wr_ref, out_ref, p_ref):
    msg = m0_ref[...] + m1_ref[...]
    pre = _dot(sph_ref[...], w1t_ref[...]) + _dot(msg, w1b_ref[...]) + b1_ref[...]
    h = _ln_relu(pre, lns_ref[...], lno_ref[...])
    upd = _dot(h, w2_ref[...]) + b2_ref[...]
    out_ref[...] = upd
    p_ref[...] = _dot(upd, wr_ref[...])


def _node(sphere, acc, w1t, w1b, b1r, lnsr, lnor, w2, b2r, wr):
    nblk = N_ACC // BN
    return pl.pallas_call(
        _node_body,
        grid=(nblk,),
        in_specs=[_row_spec(BN),
                  pl.BlockSpec((BN, D), lambda i: (i, 0)),
                  pl.BlockSpec((BN, D), lambda i: (i + N_ACC // BN, 0)),
                  _w_spec(D), _w_spec(D), _w_spec(1), _w_spec(1), _w_spec(1),
                  _w_spec(D), _w_spec(1), _w_spec(D)],
        out_specs=[_row_spec(BN), _row_spec(BN)],
        out_shape=[jax.ShapeDtypeStruct((N_ACC, D), jnp.float32)] * 2,
    )(sphere, acc, acc, w1t, w1b, b1r, lnsr, lnor, w2, b2r, wr)


def kernel(spatial_nodes, sphere_nodes, edges, senders, receivers,
           edge_W1a, edge_W1b, edge_b1, edge_ln_s, edge_ln_o, edge_W2, edge_b2,
           node_W1, node_b1, node_ln_s, node_ln_o, node_W2, node_b2):
    f32 = jnp.float32
    w1a_e, w1a_s, w1a_r = edge_W1a[:3], edge_W1a[3:3 + D], edge_W1a[3 + D:]
    w1b_e, w1b_s, w1b_r = edge_W1b[:D], edge_W1b[D:2 * D], edge_W1b[2 * D:]
    w1ae8 = jnp.zeros((8, D), f32).at[:3].set(w1a_e)
    nw1_t, nw1_b = node_W1[:D], node_W1[D:]

    edges8 = jnp.zeros((E_PAD, 8), f32).at[:E, :3].set(edges)
    send_pad = jnp.zeros((E_PAD,), jnp.int32).at[:E].set(senders)
    dummy_tgt = DUMMY + jnp.arange(E_PAD, dtype=jnp.int32) % (N_ACC - N_SPHERE)
    recv_pad = dummy_tgt.at[:E].set(receivers)
    recv3 = recv_pad.reshape(NW, PER_W // SC_CHUNK, SC_CHUNK)
    sphere_pad = jnp.zeros((N_ACC, D), f32).at[:N_SPHERE].set(sphere_nodes)
    zeros_acc = jnp.zeros((N_ACC, D), f32)

    b1r = edge_b1.reshape(1, D)
    b2r = edge_b2.reshape(1, D)
    lnsr = edge_ln_s.reshape(1, D)
    lnor = edge_ln_o.reshape(1, D)
    nb1r = node_b1.reshape(1, D)
    nb2r = node_b2.reshape(1, D)
    nlnsr = node_ln_s.reshape(1, D)
    nlnor = node_ln_o.reshape(1, D)

    s_feat = _gather(spatial_nodes, send_pad)
    proj = _proj(sphere_pad, w1a_r)
    sphere = sphere_pad
    base = bb = None
    for t in range(STEPS_):
        g = _gather_spm(proj, recv_pad)
        if t == 0:
            e_out, base, bb = _edge0(s_feat, edges8, g, w1a_s, w1ae8, w1b_s,
                                     b1r, edge_W2, w1b_e, b2r, lnsr, lnor)
        elif t < STEPS_ - 1:
            e_out, base = _edge_full(base, bb, g, edge_W2, w1b_e,
                                     b2r, lnsr, lnor)
        else:
            e_out = _edge_last(base, g, edge_W2, b2r, lnsr, lnor)
        acc = _scatter(e_out, recv3, zeros_acc)
        sphere, proj = _node(sphere, acc, nw1_t, nw1_b, nb1r, nlnsr, nlnor,
                             node_W2, nb2r, w1b_r)
    return sphere[:N_SPHERE]
